# baseline (speedup 1.0000x reference)
import functools

import jax
import jax.numpy as jnp
from jax import lax
from jax.experimental import pallas as pl
from jax.experimental.pallas import tpu as pltpu
from jax.experimental.pallas import tpu_sc as plsc

_N_NODES = 10000
_N_EDGES = 320000
_D = 128

_NC = 2
_NS = 16
_NW = _NC * _NS

_CHUNK = 512
_IDX_ROWS = _CHUNK // 128
_N_CHUNKS = _N_EDGES // _CHUNK
_HALF = _N_NODES // 2
_TRASH = _HALF
_ACC_ROWS = _HALF + 8
_ZROWS = 128
_N_ABLK = _HALF // _ZROWS
_A_TAIL = _HALF % _ZROWS


def _edge_mlp(x2d, ea_t, w1a, w1b, b1, w2, b2):
    e_tile = 2560
    grid = (_N_EDGES // e_tile,)

    def body(x_ref, ea_ref, w1a_ref, w1b_ref, b1_ref, w2_ref, b2_ref, out_ref):
        h = lax.dot_general(
            ea_ref[...],
            w1a_ref[...],
            dimension_numbers=(((0,), (0,)), ((), ())),
            preferred_element_type=jnp.float32,
        )
        h = h + jnp.dot(
            x_ref[...].astype(jnp.bfloat16),
            w1b_ref[...].astype(jnp.bfloat16),
            preferred_element_type=jnp.float32,
        )
        h = jnp.maximum(h + b1_ref[...], 0.0)
        out_ref[...] = (
            jnp.dot(
                h.astype(jnp.bfloat16),
                w2_ref[...].astype(jnp.bfloat16),
                preferred_element_type=jnp.float32,
            )
            + b2_ref[...]
        )

    return pl.pallas_call(
        body,
        grid=grid,
        in_specs=[
            pl.BlockSpec((e_tile, _D), lambda i: (i, 0)),
            pl.BlockSpec((4, e_tile), lambda i: (0, i)),
            pl.BlockSpec((4, _D), lambda i: (0, 0)),
            pl.BlockSpec((_D, _D), lambda i: (0, 0)),
            pl.BlockSpec((1, _D), lambda i: (0, 0)),
            pl.BlockSpec((_D, _D), lambda i: (0, 0)),
            pl.BlockSpec((1, _D), lambda i: (0, 0)),
        ],
        out_specs=pl.BlockSpec((e_tile, _D), lambda i: (i, 0)),
        out_shape=jax.ShapeDtypeStruct((_N_EDGES, _D), jnp.float32),
    )(x2d, ea_t, w1a, w1b, b1, w2, b2)


def _scatter_sc(vg, dst2d):
    mesh = plsc.VectorSubcoreMesh(core_axis_name="c", subcore_axis_name="s")

    @functools.partial(
        pl.kernel,
        out_type=jax.ShapeDtypeStruct((_N_NODES, _D), jnp.float32),
        mesh=mesh,
        scratch_types=[
            pltpu.VMEM((_IDX_ROWS, 128), jnp.int32),
            pltpu.VMEM((_CHUNK, _D), jnp.float32),
            pltpu.VMEM((_ZROWS, _D), jnp.float32),
            pltpu.VMEM_SHARED((_ACC_ROWS, _D), jnp.float32),
        ],
    )
    def scatter_kernel(vg_hbm, dst_hbm, out_hbm, idx_v, rows_v, zbuf, acc_sh):
        cid = lax.axis_index("c")
        sid = lax.axis_index("s")
        lo = cid * _HALF

        zv = jnp.zeros((16,), jnp.float32)

        def zrow(r, carry):
            for c16 in range(_D // 16):
                zbuf[r, pl.ds(c16 * 16, 16)] = zv
            return carry

        lax.fori_loop(0, _ZROWS, zrow, 0)

        n_blk_iters = -(-_N_ABLK // _NS)

        def zblk(it, carry):
            b = it * _NS + sid

            @pl.when(b < _N_ABLK)
            def _():
                pltpu.sync_copy(zbuf, acc_sh.at[pl.ds(b * _ZROWS, _ZROWS)])

            return carry

        lax.fori_loop(0, n_blk_iters, zblk, 0)

        @pl.when(sid == 0)
        def _():
            pltpu.sync_copy(
                zbuf.at[pl.ds(0, _A_TAIL)],
                acc_sh.at[pl.ds(_N_ABLK * _ZROWS, _A_TAIL)],
            )

        plsc.subcore_barrier()

        n_iters = -(-_N_CHUNKS // _NS)

        def chunk_body(it, carry):
            chunk = it * _NS + sid

            @pl.when(chunk < _N_CHUNKS)
            def _():
                pltpu.sync_copy(dst_hbm.at[chunk], idx_v)
                pltpu.sync_copy(vg_hbm.at[pl.ds(chunk * _CHUNK, _CHUNK)], rows_v)
                for j in range(_IDX_ROWS):
                    for c16 in range(128 // 16):
                        v = idx_v[j, pl.ds(c16 * 16, 16)]
                        local = v - lo
                        ok = (local >= 0) & (local < _HALF)
                        idx_v[j, pl.ds(c16 * 16, 16)] = jnp.where(
                            ok, local, _TRASH
                        )
                for j in range(_IDX_ROWS):
                    pltpu.sync_copy(
                        rows_v.at[pl.ds(j * 128, 128)],
                        acc_sh.at[idx_v.at[j]],
                        add=True,
                    )

            return carry

        lax.fori_loop(0, n_iters, chunk_body, 0)
        plsc.subcore_barrier()

        def wblk(it, carry):
            b = it * _NS + sid

            @pl.when(b < _N_ABLK)
            def _():
                pltpu.sync_copy(
                    acc_sh.at[pl.ds(b * _ZROWS, _ZROWS)],
                    out_hbm.at[pl.ds(lo + b * _ZROWS, _ZROWS)],
                )

            return carry

        lax.fori_loop(0, n_blk_iters, wblk, 0)

        @pl.when(sid == 0)
        def _():
            pltpu.sync_copy(
                acc_sh.at[pl.ds(_N_ABLK * _ZROWS, _A_TAIL)],
                out_hbm.at[pl.ds(lo + _N_ABLK * _ZROWS, _A_TAIL)],
            )

    return scatter_kernel(vg, dst2d)


def _node_mlp(vm, w3, b3, w4, b4):
    n_tile = 2000
    grid = (_N_NODES // n_tile,)

    def body(p_ref, w3_ref, b3_ref, w4_ref, b4_ref, out_ref):
        v = p_ref[...]
        h = jnp.maximum(
            jnp.dot(v, w3_ref[...], preferred_element_type=jnp.float32) + b3_ref[...],
            0.0,
        )
        out_ref[...] = (
            jnp.dot(h, w4_ref[...], preferred_element_type=jnp.float32) + b4_ref[...]
        )

    return pl.pallas_call(
        body,
        grid=grid,
        in_specs=[
            pl.BlockSpec((n_tile, _D), lambda i: (i, 0)),
            pl.BlockSpec((_D, _D), lambda i: (0, 0)),
            pl.BlockSpec((1, _D), lambda i: (0, 0)),
            pl.BlockSpec((_D, _D), lambda i: (0, 0)),
            pl.BlockSpec((1, _D), lambda i: (0, 0)),
        ],
        out_specs=pl.BlockSpec((n_tile, _D), lambda i: (i, 0)),
        out_shape=jax.ShapeDtypeStruct((_N_NODES, _D), jnp.float32),
    )(vm, w3, b3, w4, b4)


def kernel(x, edge_index, edge_attr, W1, b1, W2, b2, W3, b3, W4, b4):
    x2d = x.reshape(_N_EDGES, _D)
    w1a = W1[:4]
    w1b = W1[4:]
    vg = _edge_mlp(
        x2d,
        edge_attr.T,
        w1a,
        w1b,
        b1.reshape(1, _D),
        W2,
        b2.reshape(1, _D),
    )
    dst3d = edge_index[1].reshape(_N_CHUNKS, _IDX_ROWS, 128)
    vm = _scatter_sc(vg, dst3d)
    out = _node_mlp(vm, W3, b3.reshape(1, _D), W4, b4.reshape(1, _D))
    return out.reshape(1, _N_NODES, _D)


# speedup vs baseline: 1.0943x; 1.0943x over previous
import functools

import jax
import jax.numpy as jnp
from jax import lax
from jax.experimental import pallas as pl
from jax.experimental.pallas import tpu as pltpu
from jax.experimental.pallas import tpu_sc as plsc

_N_NODES = 10000
_N_EDGES = 320000
_D = 128

_NC = 2
_NS = 16
_NW = _NC * _NS

_CHUNK = 512
_IDX_ROWS = _CHUNK // 128
_N_CHUNKS = _N_EDGES // _CHUNK
_HALF = _N_NODES // 2
_TRASH = _HALF
_ACC_ROWS = _HALF + 16
_ZROWS = 128
_N_ABLK = _HALF // _ZROWS
_A_TAIL = _HALF % _ZROWS


def _edge_mlp(x2d, ea_t, w1a, w1b, b1, w2, b2):
    e_tile = 2560
    grid = (_N_EDGES // e_tile,)

    def body(x_ref, ea_ref, w1a_ref, w1b_ref, b1_ref, w2_ref, b2_ref, out_ref):
        h = lax.dot_general(
            ea_ref[...],
            w1a_ref[...],
            dimension_numbers=(((0,), (0,)), ((), ())),
            preferred_element_type=jnp.float32,
        )
        h = h + jnp.dot(
            x_ref[...].astype(jnp.bfloat16),
            w1b_ref[...].astype(jnp.bfloat16),
            preferred_element_type=jnp.float32,
        )
        h = jnp.maximum(h + b1_ref[...], 0.0)
        out_ref[...] = (
            jnp.dot(
                h.astype(jnp.bfloat16),
                w2_ref[...].astype(jnp.bfloat16),
                preferred_element_type=jnp.float32,
            )
            + b2_ref[...]
        )

    return pl.pallas_call(
        body,
        grid=grid,
        in_specs=[
            pl.BlockSpec((e_tile, _D), lambda i: (i, 0)),
            pl.BlockSpec((4, e_tile), lambda i: (0, i)),
            pl.BlockSpec((4, _D), lambda i: (0, 0)),
            pl.BlockSpec((_D, _D), lambda i: (0, 0)),
            pl.BlockSpec((1, _D), lambda i: (0, 0)),
            pl.BlockSpec((_D, _D), lambda i: (0, 0)),
            pl.BlockSpec((1, _D), lambda i: (0, 0)),
        ],
        out_specs=pl.BlockSpec((e_tile, _D), lambda i: (i, 0)),
        out_shape=jax.ShapeDtypeStruct((_N_EDGES, _D), jnp.float32),
    )(x2d, ea_t, w1a, w1b, b1, w2, b2)


def _scatter_sc(vg, dst2d):
    mesh = plsc.VectorSubcoreMesh(core_axis_name="c", subcore_axis_name="s")

    @functools.partial(
        pl.kernel,
        out_type=jax.ShapeDtypeStruct((_N_NODES, _D), jnp.float32),
        mesh=mesh,
        scratch_types=[
            pltpu.VMEM((_IDX_ROWS, 128), jnp.int32),
            pltpu.VMEM((_CHUNK, _D), jnp.float32),
            pltpu.VMEM((_ZROWS, _D), jnp.float32),
            pltpu.VMEM_SHARED((_ACC_ROWS, _D), jnp.float32),
        ],
    )
    def scatter_kernel(vg_hbm, dst_hbm, out_hbm, idx_v, rows_v, zbuf, acc_sh):
        cid = lax.axis_index("c")
        sid = lax.axis_index("s")
        lo = cid * _HALF

        zv = jnp.zeros((16,), jnp.float32)

        def zrow(r, carry):
            for c16 in range(_D // 16):
                zbuf[r, pl.ds(c16 * 16, 16)] = zv
            return carry

        lax.fori_loop(0, _ZROWS, zrow, 0)

        n_blk_iters = -(-_N_ABLK // _NS)

        def zblk(it, carry):
            b = it * _NS + sid

            @pl.when(b < _N_ABLK)
            def _():
                pltpu.sync_copy(zbuf, acc_sh.at[pl.ds(b * _ZROWS, _ZROWS)])

            return carry

        lax.fori_loop(0, n_blk_iters, zblk, 0)

        @pl.when(sid == 0)
        def _():
            pltpu.sync_copy(
                zbuf.at[pl.ds(0, _A_TAIL)],
                acc_sh.at[pl.ds(_N_ABLK * _ZROWS, _A_TAIL)],
            )

        plsc.subcore_barrier()

        n_iters = -(-_N_CHUNKS // _NS)

        def chunk_body(it, carry):
            chunk = it * _NS + sid

            @pl.when(chunk < _N_CHUNKS)
            def _():
                pltpu.sync_copy(dst_hbm.at[chunk], idx_v)
                pltpu.sync_copy(vg_hbm.at[pl.ds(chunk * _CHUNK, _CHUNK)], rows_v)
                trash = _TRASH + lax.iota(jnp.int32, 16)
                for j in range(_IDX_ROWS):
                    for c16 in range(128 // 16):
                        v = idx_v[j, pl.ds(c16 * 16, 16)]
                        local = v - lo
                        ok = (local >= 0) & (local < _HALF)
                        idx_v[j, pl.ds(c16 * 16, 16)] = jnp.where(
                            ok, local, trash
                        )
                for j in range(_IDX_ROWS):
                    pltpu.sync_copy(
                        rows_v.at[pl.ds(j * 128, 128)],
                        acc_sh.at[idx_v.at[j]],
                        add=True,
                    )

            return carry

        lax.fori_loop(0, n_iters, chunk_body, 0)
        plsc.subcore_barrier()

        def wblk(it, carry):
            b = it * _NS + sid

            @pl.when(b < _N_ABLK)
            def _():
                pltpu.sync_copy(
                    acc_sh.at[pl.ds(b * _ZROWS, _ZROWS)],
                    out_hbm.at[pl.ds(lo + b * _ZROWS, _ZROWS)],
                )

            return carry

        lax.fori_loop(0, n_blk_iters, wblk, 0)

        @pl.when(sid == 0)
        def _():
            pltpu.sync_copy(
                acc_sh.at[pl.ds(_N_ABLK * _ZROWS, _A_TAIL)],
                out_hbm.at[pl.ds(lo + _N_ABLK * _ZROWS, _A_TAIL)],
            )

    return scatter_kernel(vg, dst2d)


def _node_mlp(vm, w3, b3, w4, b4):
    n_tile = 2000
    grid = (_N_NODES // n_tile,)

    def body(p_ref, w3_ref, b3_ref, w4_ref, b4_ref, out_ref):
        v = p_ref[...]
        h = jnp.maximum(
            jnp.dot(v, w3_ref[...], preferred_element_type=jnp.float32) + b3_ref[...],
            0.0,
        )
        out_ref[...] = (
            jnp.dot(h, w4_ref[...], preferred_element_type=jnp.float32) + b4_ref[...]
        )

    return pl.pallas_call(
        body,
        grid=grid,
        in_specs=[
            pl.BlockSpec((n_tile, _D), lambda i: (i, 0)),
            pl.BlockSpec((_D, _D), lambda i: (0, 0)),
            pl.BlockSpec((1, _D), lambda i: (0, 0)),
            pl.BlockSpec((_D, _D), lambda i: (0, 0)),
            pl.BlockSpec((1, _D), lambda i: (0, 0)),
        ],
        out_specs=pl.BlockSpec((n_tile, _D), lambda i: (i, 0)),
        out_shape=jax.ShapeDtypeStruct((_N_NODES, _D), jnp.float32),
    )(vm, w3, b3, w4, b4)


def kernel(x, edge_index, edge_attr, W1, b1, W2, b2, W3, b3, W4, b4):
    x2d = x.reshape(_N_EDGES, _D)
    w1a = W1[:4]
    w1b = W1[4:]
    vg = _edge_mlp(
        x2d,
        edge_attr.T,
        w1a,
        w1b,
        b1.reshape(1, _D),
        W2,
        b2.reshape(1, _D),
    )
    dst3d = edge_index[1].reshape(_N_CHUNKS, _IDX_ROWS, 128)
    vm = _scatter_sc(vg, dst3d)
    out = _node_mlp(vm, W3, b3.reshape(1, _D), W4, b4.reshape(1, _D))
    return out.reshape(1, _N_NODES, _D)


# speedup vs baseline: 1.3390x; 1.2236x over previous
import functools

import jax
import jax.numpy as jnp
from jax import lax
from jax.experimental import pallas as pl
from jax.experimental.pallas import tpu as pltpu
from jax.experimental.pallas import tpu_sc as plsc

_N_NODES = 10000
_N_EDGES = 320000
_D = 128

_NC = 2
_NS = 16
_NW = _NC * _NS

_CHUNK = 256
_IDX_ROWS = _CHUNK // 128
_N_CHUNKS = _N_EDGES // _CHUNK
_NBUF = 2
_HALF = _N_NODES // 2
_TRASH = _HALF
_ACC_ROWS = _HALF + 16
_ZROWS = 128
_N_ABLK = _HALF // _ZROWS
_A_TAIL = _HALF % _ZROWS


def _edge_mlp(x2d, ea_t, w1a, w1b, b1, w2, b2):
    e_tile = 2560
    grid = (_N_EDGES // e_tile,)

    def body(x_ref, ea_ref, w1a_ref, w1b_ref, b1_ref, w2_ref, b2_ref, out_ref):
        h = lax.dot_general(
            ea_ref[...],
            w1a_ref[...],
            dimension_numbers=(((0,), (0,)), ((), ())),
            preferred_element_type=jnp.float32,
        )
        h = h + jnp.dot(
            x_ref[...].astype(jnp.bfloat16),
            w1b_ref[...].astype(jnp.bfloat16),
            preferred_element_type=jnp.float32,
        )
        h = jnp.maximum(h + b1_ref[...], 0.0)
        out_ref[...] = (
            jnp.dot(
                h.astype(jnp.bfloat16),
                w2_ref[...].astype(jnp.bfloat16),
                preferred_element_type=jnp.float32,
            )
            + b2_ref[...]
        )

    return pl.pallas_call(
        body,
        grid=grid,
        in_specs=[
            pl.BlockSpec((e_tile, _D), lambda i: (i, 0)),
            pl.BlockSpec((4, e_tile), lambda i: (0, i)),
            pl.BlockSpec((4, _D), lambda i: (0, 0)),
            pl.BlockSpec((_D, _D), lambda i: (0, 0)),
            pl.BlockSpec((1, _D), lambda i: (0, 0)),
            pl.BlockSpec((_D, _D), lambda i: (0, 0)),
            pl.BlockSpec((1, _D), lambda i: (0, 0)),
        ],
        out_specs=pl.BlockSpec((e_tile, _D), lambda i: (i, 0)),
        out_shape=jax.ShapeDtypeStruct((_N_EDGES, _D), jnp.float32),
    )(x2d, ea_t, w1a, w1b, b1, w2, b2)


def _scatter_sc(vg, dst2d):
    mesh = plsc.VectorSubcoreMesh(core_axis_name="c", subcore_axis_name="s")

    @functools.partial(
        pl.kernel,
        out_type=jax.ShapeDtypeStruct((_N_NODES, _D), jnp.float32),
        mesh=mesh,
        scratch_types=[
            pltpu.VMEM((_NBUF, _IDX_ROWS, 128), jnp.int32),
            pltpu.VMEM((_NBUF, _CHUNK, _D), jnp.float32),
            pltpu.VMEM_SHARED((_ACC_ROWS, _D), jnp.float32),
            pltpu.SemaphoreType.DMA,
            pltpu.SemaphoreType.DMA,
            pltpu.SemaphoreType.DMA,
            pltpu.SemaphoreType.DMA,
            pltpu.SemaphoreType.DMA,
            pltpu.SemaphoreType.DMA,
        ],
    )
    def scatter_kernel(
        vg_hbm, dst_hbm, out_hbm, idx_v, rows_v, acc_sh, r0, r1, r2, s0, s1, s2
    ):
        cid = lax.axis_index("c")
        sid = lax.axis_index("s")
        lo = cid * _HALF
        rsem = [r0, r1, r2]
        ssem = [s0, s1, s2]

        zbuf = rows_v.at[0].at[pl.ds(0, _ZROWS)]
        zv = jnp.zeros((16,), jnp.float32)

        def zrow(r, carry):
            for c16 in range(_D // 16):
                rows_v[0, r, pl.ds(c16 * 16, 16)] = zv
            return carry

        lax.fori_loop(0, _ZROWS, zrow, 0)

        n_blk_iters = -(-_N_ABLK // _NS)

        def zblk(it, carry):
            b = it * _NS + sid

            @pl.when(b < _N_ABLK)
            def _():
                pltpu.sync_copy(zbuf, acc_sh.at[pl.ds(b * _ZROWS, _ZROWS)])

            return carry

        lax.fori_loop(0, n_blk_iters, zblk, 0)

        @pl.when(sid == 0)
        def _():
            pltpu.sync_copy(
                zbuf.at[pl.ds(0, _A_TAIL)],
                acc_sh.at[pl.ds(_N_ABLK * _ZROWS, _A_TAIL)],
            )

        n_i = jnp.where(sid < 2, _N_CHUNKS // _NS + 1, _N_CHUNKS // _NS)

        def issue_reads(k, b):
            chunk = k * _NS + sid
            pltpu.async_copy(dst_hbm.at[chunk], idx_v.at[b], rsem[b])
            pltpu.async_copy(
                vg_hbm.at[pl.ds(chunk * _CHUNK, _CHUNK)], rows_v.at[b], rsem[b]
            )

        def wait_reads(k, b):
            chunk = k * _NS + sid
            pltpu.make_async_copy(dst_hbm.at[chunk], idx_v.at[b], rsem[b]).wait()
            pltpu.make_async_copy(
                vg_hbm.at[pl.ds(chunk * _CHUNK, _CHUNK)], rows_v.at[b], rsem[b]
            ).wait()

        def drain_adds(b):
            for j in range(_IDX_ROWS):
                pltpu.make_async_copy(
                    rows_v.at[b].at[pl.ds(j * 128, 128)],
                    acc_sh.at[idx_v.at[b].at[j]],
                    ssem[b],
                ).wait()

        issue_reads(0, 0)
        issue_reads(1, 1)
        plsc.subcore_barrier()

        trash = _TRASH + lax.iota(jnp.int32, 16)
        n_slots = -(-(_N_CHUNKS // _NS + 1) // _NBUF)

        def slot_body(it3, carry):
            for b in range(_NBUF):
                k = it3 * _NBUF + b
                b2 = (b + 2) % _NBUF

                @pl.when(k < n_i)
                def _():
                    wait_reads(k, b)
                    for j in range(_IDX_ROWS):
                        for c16 in range(128 // 16):
                            v = idx_v[b, j, pl.ds(c16 * 16, 16)]
                            local = v - lo
                            ok = (local >= 0) & (local < _HALF)
                            idx_v[b, j, pl.ds(c16 * 16, 16)] = jnp.where(
                                ok, local, trash
                            )
                    for j in range(_IDX_ROWS):
                        pltpu.async_copy(
                            rows_v.at[b].at[pl.ds(j * 128, 128)],
                            acc_sh.at[idx_v.at[b].at[j]],
                            ssem[b],
                            add=True,
                        )

                drain_guard = (
                    (k + 2 < n_i) if _NBUF == 2 else (k >= 1) & (k + 2 < n_i)
                )

                @pl.when(drain_guard)
                def _():
                    drain_adds(b2)

                @pl.when(k + 2 < n_i)
                def _():
                    issue_reads(k + 2, b2)

            return carry

        lax.fori_loop(0, n_slots, slot_body, 0)
        for b in range(_NBUF):
            drain_adds(b)
        plsc.subcore_barrier()

        def wblk(it, carry):
            b = it * _NS + sid

            @pl.when(b < _N_ABLK)
            def _():
                pltpu.sync_copy(
                    acc_sh.at[pl.ds(b * _ZROWS, _ZROWS)],
                    out_hbm.at[pl.ds(lo + b * _ZROWS, _ZROWS)],
                )

            return carry

        lax.fori_loop(0, n_blk_iters, wblk, 0)

        @pl.when(sid == 0)
        def _():
            pltpu.sync_copy(
                acc_sh.at[pl.ds(_N_ABLK * _ZROWS, _A_TAIL)],
                out_hbm.at[pl.ds(lo + _N_ABLK * _ZROWS, _A_TAIL)],
            )

    return scatter_kernel(vg, dst2d)


def _node_mlp(vm, w3, b3, w4, b4):
    n_tile = 2000
    grid = (_N_NODES // n_tile,)

    def body(p_ref, w3_ref, b3_ref, w4_ref, b4_ref, out_ref):
        v = p_ref[...]
        h = jnp.maximum(
            jnp.dot(v, w3_ref[...], preferred_element_type=jnp.float32) + b3_ref[...],
            0.0,
        )
        out_ref[...] = (
            jnp.dot(h, w4_ref[...], preferred_element_type=jnp.float32) + b4_ref[...]
        )

    return pl.pallas_call(
        body,
        grid=grid,
        in_specs=[
            pl.BlockSpec((n_tile, _D), lambda i: (i, 0)),
            pl.BlockSpec((_D, _D), lambda i: (0, 0)),
            pl.BlockSpec((1, _D), lambda i: (0, 0)),
            pl.BlockSpec((_D, _D), lambda i: (0, 0)),
            pl.BlockSpec((1, _D), lambda i: (0, 0)),
        ],
        out_specs=pl.BlockSpec((n_tile, _D), lambda i: (i, 0)),
        out_shape=jax.ShapeDtypeStruct((_N_NODES, _D), jnp.float32),
    )(vm, w3, b3, w4, b4)


def kernel(x, edge_index, edge_attr, W1, b1, W2, b2, W3, b3, W4, b4):
    x2d = x.reshape(_N_EDGES, _D)
    w1a = W1[:4]
    w1b = W1[4:]
    vg = _edge_mlp(
        x2d,
        edge_attr.T,
        w1a,
        w1b,
        b1.reshape(1, _D),
        W2,
        b2.reshape(1, _D),
    )
    dst3d = edge_index[1].reshape(_N_CHUNKS, _IDX_ROWS, 128)
    vm = _scatter_sc(vg, dst3d)
    out = _node_mlp(vm, W3, b3.reshape(1, _D), W4, b4.reshape(1, _D))
    return out.reshape(1, _N_NODES, _D)


# speedup vs baseline: 1.6222x; 1.2116x over previous
import functools

import jax
import jax.numpy as jnp
from jax import lax
from jax.experimental import pallas as pl
from jax.experimental.pallas import tpu as pltpu
from jax.experimental.pallas import tpu_sc as plsc

_N_NODES = 10000
_N_EDGES = 320000
_D = 128

_NC = 2
_NS = 16
_NW = _NC * _NS

_N_SLICES = 2
_E_HALF = _N_EDGES // _N_SLICES
_CHUNK = 256
_IDX_ROWS = _CHUNK // 128
_N_CHUNKS = _E_HALF // _CHUNK
_NBUF = 2
_HALF = _N_NODES // 2
_TRASH = _HALF
_ACC_ROWS = _HALF + 16
_ZROWS = 128
_N_ABLK = _HALF // _ZROWS
_A_TAIL = _HALF % _ZROWS


def _edge_mlp(x2d, ea_t, w1a, w1b, b1, w2, b2, half):
    e_tile = 3200
    grid = (_E_HALF // e_tile,)
    off = half * (_E_HALF // e_tile)

    def body(x_ref, ea_ref, w1a_ref, w1b_ref, b1_ref, w2_ref, b2_ref, out_ref):
        h = lax.dot_general(
            ea_ref[...],
            w1a_ref[...],
            dimension_numbers=(((0,), (0,)), ((), ())),
            preferred_element_type=jnp.float32,
        )
        h = h + jnp.dot(
            x_ref[...].astype(jnp.bfloat16),
            w1b_ref[...].astype(jnp.bfloat16),
            preferred_element_type=jnp.float32,
        )
        h = jnp.maximum(h + b1_ref[...], 0.0)
        out_ref[...] = (
            jnp.dot(
                h.astype(jnp.bfloat16),
                w2_ref[...].astype(jnp.bfloat16),
                preferred_element_type=jnp.float32,
            )
            + b2_ref[...]
        )

    return pl.pallas_call(
        body,
        grid=grid,
        in_specs=[
            pl.BlockSpec((e_tile, _D), lambda i: (i + off, 0)),
            pl.BlockSpec((4, e_tile), lambda i: (0, i + off)),
            pl.BlockSpec((4, _D), lambda i: (0, 0)),
            pl.BlockSpec((_D, _D), lambda i: (0, 0)),
            pl.BlockSpec((1, _D), lambda i: (0, 0)),
            pl.BlockSpec((_D, _D), lambda i: (0, 0)),
            pl.BlockSpec((1, _D), lambda i: (0, 0)),
        ],
        out_specs=pl.BlockSpec((e_tile, _D), lambda i: (i, 0)),
        out_shape=jax.ShapeDtypeStruct((_E_HALF, _D), jnp.float32),
    )(x2d, ea_t, w1a, w1b, b1, w2, b2)


def _scatter_sc(vg, dst2d):
    mesh = plsc.VectorSubcoreMesh(core_axis_name="c", subcore_axis_name="s")

    @functools.partial(
        pl.kernel,
        out_type=jax.ShapeDtypeStruct((_N_NODES, _D), jnp.float32),
        mesh=mesh,
        scratch_types=[
            pltpu.VMEM((_NBUF, _IDX_ROWS, 128), jnp.int32),
            pltpu.VMEM((_NBUF, _CHUNK, _D), jnp.float32),
            pltpu.VMEM_SHARED((_ACC_ROWS, _D), jnp.float32),
            pltpu.SemaphoreType.DMA,
            pltpu.SemaphoreType.DMA,
            pltpu.SemaphoreType.DMA,
            pltpu.SemaphoreType.DMA,
            pltpu.SemaphoreType.DMA,
            pltpu.SemaphoreType.DMA,
        ],
    )
    def scatter_kernel(
        vg_hbm, dst_hbm, out_hbm, idx_v, rows_v, acc_sh, r0, r1, r2, s0, s1, s2
    ):
        cid = lax.axis_index("c")
        sid = lax.axis_index("s")
        lo = cid * _HALF
        rsem = [r0, r1, r2]
        ssem = [s0, s1, s2]

        zbuf = rows_v.at[0].at[pl.ds(0, _ZROWS)]
        zv = jnp.zeros((16,), jnp.float32)

        def zrow(r, carry):
            for c16 in range(_D // 16):
                rows_v[0, r, pl.ds(c16 * 16, 16)] = zv
            return carry

        lax.fori_loop(0, _ZROWS, zrow, 0)

        n_blk_iters = -(-_N_ABLK // _NS)

        def zblk(it, carry):
            b = it * _NS + sid

            @pl.when(b < _N_ABLK)
            def _():
                pltpu.sync_copy(zbuf, acc_sh.at[pl.ds(b * _ZROWS, _ZROWS)])

            return carry

        lax.fori_loop(0, n_blk_iters, zblk, 0)

        @pl.when(sid == 0)
        def _():
            pltpu.sync_copy(
                zbuf.at[pl.ds(0, _A_TAIL)],
                acc_sh.at[pl.ds(_N_ABLK * _ZROWS, _A_TAIL)],
            )

        n_extra = _N_CHUNKS - (_N_CHUNKS // _NS) * _NS
        n_i = jnp.where(sid < n_extra, _N_CHUNKS // _NS + 1, _N_CHUNKS // _NS)

        def issue_reads(k, b):
            chunk = k * _NS + sid
            pltpu.async_copy(dst_hbm.at[chunk], idx_v.at[b], rsem[b])
            pltpu.async_copy(
                vg_hbm.at[pl.ds(chunk * _CHUNK, _CHUNK)], rows_v.at[b], rsem[b]
            )

        def wait_reads(k, b):
            chunk = k * _NS + sid
            pltpu.make_async_copy(dst_hbm.at[chunk], idx_v.at[b], rsem[b]).wait()
            pltpu.make_async_copy(
                vg_hbm.at[pl.ds(chunk * _CHUNK, _CHUNK)], rows_v.at[b], rsem[b]
            ).wait()

        def drain_adds(b):
            for j in range(_IDX_ROWS):
                pltpu.make_async_copy(
                    rows_v.at[b].at[pl.ds(j * 128, 128)],
                    acc_sh.at[idx_v.at[b].at[j]],
                    ssem[b],
                ).wait()

        issue_reads(0, 0)
        issue_reads(1, 1)
        plsc.subcore_barrier()

        trash = _TRASH + lax.iota(jnp.int32, 16)
        n_slots = -(-(_N_CHUNKS // _NS + 1) // _NBUF)

        def slot_body(it3, carry):
            for b in range(_NBUF):
                k = it3 * _NBUF + b
                b2 = (b + 2) % _NBUF

                @pl.when(k < n_i)
                def _():
                    wait_reads(k, b)
                    for j in range(_IDX_ROWS):
                        for c16 in range(128 // 16):
                            v = idx_v[b, j, pl.ds(c16 * 16, 16)]
                            local = v - lo
                            ok = (local >= 0) & (local < _HALF)
                            idx_v[b, j, pl.ds(c16 * 16, 16)] = jnp.where(
                                ok, local, trash
                            )
                    for j in range(_IDX_ROWS):
                        pltpu.async_copy(
                            rows_v.at[b].at[pl.ds(j * 128, 128)],
                            acc_sh.at[idx_v.at[b].at[j]],
                            ssem[b],
                            add=True,
                        )

                drain_guard = (
                    (k + 2 < n_i) if _NBUF == 2 else (k >= 1) & (k + 2 < n_i)
                )

                @pl.when(drain_guard)
                def _():
                    drain_adds(b2)

                @pl.when(k + 2 < n_i)
                def _():
                    issue_reads(k + 2, b2)

            return carry

        lax.fori_loop(0, n_slots, slot_body, 0)
        for b in range(_NBUF):
            drain_adds(b)
        plsc.subcore_barrier()

        def wblk(it, carry):
            b = it * _NS + sid

            @pl.when(b < _N_ABLK)
            def _():
                pltpu.sync_copy(
                    acc_sh.at[pl.ds(b * _ZROWS, _ZROWS)],
                    out_hbm.at[pl.ds(lo + b * _ZROWS, _ZROWS)],
                )

            return carry

        lax.fori_loop(0, n_blk_iters, wblk, 0)

        @pl.when(sid == 0)
        def _():
            pltpu.sync_copy(
                acc_sh.at[pl.ds(_N_ABLK * _ZROWS, _A_TAIL)],
                out_hbm.at[pl.ds(lo + _N_ABLK * _ZROWS, _A_TAIL)],
            )

    return scatter_kernel(vg, dst2d)


def _node_mlp(parts, w3, b3, w4, b4):
    n_tile = 2000
    grid = (_N_NODES // n_tile,)

    def body(p0_ref, p1_ref, w3_ref, b3_ref, w4_ref, b4_ref, out_ref):
        v = p0_ref[...] + p1_ref[...]
        h = jnp.maximum(
            jnp.dot(v, w3_ref[...], preferred_element_type=jnp.float32) + b3_ref[...],
            0.0,
        )
        out_ref[...] = (
            jnp.dot(h, w4_ref[...], preferred_element_type=jnp.float32) + b4_ref[...]
        )

    return pl.pallas_call(
        body,
        grid=grid,
        in_specs=[
            pl.BlockSpec((n_tile, _D), lambda i: (i, 0)),
            pl.BlockSpec((n_tile, _D), lambda i: (i, 0)),
            pl.BlockSpec((_D, _D), lambda i: (0, 0)),
            pl.BlockSpec((1, _D), lambda i: (0, 0)),
            pl.BlockSpec((_D, _D), lambda i: (0, 0)),
            pl.BlockSpec((1, _D), lambda i: (0, 0)),
        ],
        out_specs=pl.BlockSpec((n_tile, _D), lambda i: (i, 0)),
        out_shape=jax.ShapeDtypeStruct((_N_NODES, _D), jnp.float32),
    )(parts[0], parts[1], w3, b3, w4, b4)


def kernel(x, edge_index, edge_attr, W1, b1, W2, b2, W3, b3, W4, b4):
    x2d = x.reshape(_N_EDGES, _D)
    w1a = W1[:4]
    w1b = W1[4:]
    vg = [
        _edge_mlp(
            x2d,
            edge_attr.T,
            w1a,
            w1b,
            b1.reshape(1, _D),
            W2,
            b2.reshape(1, _D),
            h,
        )
        for h in range(_N_SLICES)
    ]
    dst4 = edge_index[1].reshape(_N_SLICES, _N_CHUNKS, _IDX_ROWS, 128)
    parts = [_scatter_sc(vg[h], dst4[h]) for h in range(_N_SLICES)]
    out = _node_mlp(parts, W3, b3.reshape(1, _D), W4, b4.reshape(1, _D))
    return out.reshape(1, _N_NODES, _D)


# speedup vs baseline: 1.6574x; 1.0217x over previous
import functools

import jax
import jax.numpy as jnp
from jax import lax
from jax.experimental import pallas as pl
from jax.experimental.pallas import tpu as pltpu
from jax.experimental.pallas import tpu_sc as plsc

_N_NODES = 10000
_N_EDGES = 320000
_D = 128

_NC = 2
_NS = 16
_NW = _NC * _NS

_N_SLICES = 2
_E_HALF = _N_EDGES // _N_SLICES
_CHUNK = 256
_IDX_ROWS = _CHUNK // 128
_N_CHUNKS = _E_HALF // _CHUNK
_NBUF = 2
_HALF = _N_NODES // 2
_TRASH = _HALF
_ACC_ROWS = _HALF + 16
_ZROWS = 128
_N_ABLK = _HALF // _ZROWS
_A_TAIL = _HALF % _ZROWS


def _edge_mlp(x2d, ea_t, w1a, w1b, b1, w2, b2, half):
    e_tile = 6400
    grid = (_E_HALF // e_tile,)
    off = half * (_E_HALF // e_tile)

    def body(x_ref, ea_ref, w1a_ref, w1b_ref, b1_ref, w2_ref, b2_ref, out_ref):
        h = lax.dot_general(
            ea_ref[...],
            w1a_ref[...],
            dimension_numbers=(((0,), (0,)), ((), ())),
            preferred_element_type=jnp.float32,
        )
        h = h + jnp.dot(
            x_ref[...].astype(jnp.bfloat16),
            w1b_ref[...].astype(jnp.bfloat16),
            preferred_element_type=jnp.float32,
        )
        h = jnp.maximum(h + b1_ref[...], 0.0)
        out_ref[...] = (
            jnp.dot(
                h.astype(jnp.bfloat16),
                w2_ref[...].astype(jnp.bfloat16),
                preferred_element_type=jnp.float32,
            )
            + b2_ref[...]
        )

    return pl.pallas_call(
        body,
        grid=grid,
        in_specs=[
            pl.BlockSpec((e_tile, _D), lambda i: (i + off, 0)),
            pl.BlockSpec((4, e_tile), lambda i: (0, i + off)),
            pl.BlockSpec((4, _D), lambda i: (0, 0)),
            pl.BlockSpec((_D, _D), lambda i: (0, 0)),
            pl.BlockSpec((1, _D), lambda i: (0, 0)),
            pl.BlockSpec((_D, _D), lambda i: (0, 0)),
            pl.BlockSpec((1, _D), lambda i: (0, 0)),
        ],
        out_specs=pl.BlockSpec((e_tile, _D), lambda i: (i, 0)),
        out_shape=jax.ShapeDtypeStruct((_E_HALF, _D), jnp.float32),
    )(x2d, ea_t, w1a, w1b, b1, w2, b2)


def _scatter_sc(vg, dst2d):
    mesh = plsc.VectorSubcoreMesh(core_axis_name="c", subcore_axis_name="s")

    @functools.partial(
        pl.kernel,
        out_type=jax.ShapeDtypeStruct((_N_NODES, _D), jnp.float32),
        mesh=mesh,
        scratch_types=[
            pltpu.VMEM((_NBUF, _IDX_ROWS, 128), jnp.int32),
            pltpu.VMEM((_NBUF, _CHUNK, _D), jnp.float32),
            pltpu.VMEM_SHARED((_ACC_ROWS, _D), jnp.float32),
            pltpu.SemaphoreType.DMA,
            pltpu.SemaphoreType.DMA,
            pltpu.SemaphoreType.DMA,
            pltpu.SemaphoreType.DMA,
            pltpu.SemaphoreType.DMA,
            pltpu.SemaphoreType.DMA,
        ],
    )
    def scatter_kernel(
        vg_hbm, dst_hbm, out_hbm, idx_v, rows_v, acc_sh, r0, r1, r2, s0, s1, s2
    ):
        cid = lax.axis_index("c")
        sid = lax.axis_index("s")
        lo = cid * _HALF
        rsem = [r0, r1, r2]
        ssem = [s0, s1, s2]

        zbuf = rows_v.at[0].at[pl.ds(0, _ZROWS)]
        zv = jnp.zeros((16,), jnp.float32)

        def zrow(r, carry):
            for c16 in range(_D // 16):
                rows_v[0, r, pl.ds(c16 * 16, 16)] = zv
            return carry

        lax.fori_loop(0, _ZROWS, zrow, 0)

        n_blk_iters = -(-_N_ABLK // _NS)

        def zblk(it, carry):
            b = it * _NS + sid

            @pl.when(b < _N_ABLK)
            def _():
                pltpu.sync_copy(zbuf, acc_sh.at[pl.ds(b * _ZROWS, _ZROWS)])

            return carry

        lax.fori_loop(0, n_blk_iters, zblk, 0)

        @pl.when(sid == 0)
        def _():
            pltpu.sync_copy(
                zbuf.at[pl.ds(0, _A_TAIL)],
                acc_sh.at[pl.ds(_N_ABLK * _ZROWS, _A_TAIL)],
            )

        n_extra = _N_CHUNKS - (_N_CHUNKS // _NS) * _NS
        n_i = jnp.where(sid < n_extra, _N_CHUNKS // _NS + 1, _N_CHUNKS // _NS)

        def issue_reads(k, b):
            chunk = k * _NS + sid
            pltpu.async_copy(dst_hbm.at[chunk], idx_v.at[b], rsem[b])
            pltpu.async_copy(
                vg_hbm.at[pl.ds(chunk * _CHUNK, _CHUNK)], rows_v.at[b], rsem[b]
            )

        def wait_reads(k, b):
            chunk = k * _NS + sid
            pltpu.make_async_copy(dst_hbm.at[chunk], idx_v.at[b], rsem[b]).wait()
            pltpu.make_async_copy(
                vg_hbm.at[pl.ds(chunk * _CHUNK, _CHUNK)], rows_v.at[b], rsem[b]
            ).wait()

        def drain_adds(b):
            for j in range(_IDX_ROWS):
                pltpu.make_async_copy(
                    rows_v.at[b].at[pl.ds(j * 128, 128)],
                    acc_sh.at[idx_v.at[b].at[j]],
                    ssem[b],
                ).wait()

        issue_reads(0, 0)
        issue_reads(1, 1)
        plsc.subcore_barrier()

        trash = _TRASH + lax.iota(jnp.int32, 16)
        n_slots = -(-(_N_CHUNKS // _NS + 1) // _NBUF)

        def slot_body(it3, carry):
            for b in range(_NBUF):
                k = it3 * _NBUF + b
                b2 = (b + 2) % _NBUF

                @pl.when(k < n_i)
                def _():
                    wait_reads(k, b)
                    for j in range(_IDX_ROWS):
                        for c16 in range(128 // 16):
                            v = idx_v[b, j, pl.ds(c16 * 16, 16)]
                            local = v - lo
                            ok = (local >= 0) & (local < _HALF)
                            idx_v[b, j, pl.ds(c16 * 16, 16)] = jnp.where(
                                ok, local, trash
                            )
                    for j in range(_IDX_ROWS):
                        pltpu.async_copy(
                            rows_v.at[b].at[pl.ds(j * 128, 128)],
                            acc_sh.at[idx_v.at[b].at[j]],
                            ssem[b],
                            add=True,
                        )

                drain_guard = (
                    (k + 2 < n_i) if _NBUF == 2 else (k >= 1) & (k + 2 < n_i)
                )

                @pl.when(drain_guard)
                def _():
                    drain_adds(b2)

                @pl.when(k + 2 < n_i)
                def _():
                    issue_reads(k + 2, b2)

            return carry

        lax.fori_loop(0, n_slots, slot_body, 0)
        for b in range(_NBUF):
            drain_adds(b)
        plsc.subcore_barrier()

        def wblk(it, carry):
            b = it * _NS + sid

            @pl.when(b < _N_ABLK)
            def _():
                pltpu.sync_copy(
                    acc_sh.at[pl.ds(b * _ZROWS, _ZROWS)],
                    out_hbm.at[pl.ds(lo + b * _ZROWS, _ZROWS)],
                )

            return carry

        lax.fori_loop(0, n_blk_iters, wblk, 0)

        @pl.when(sid == 0)
        def _():
            pltpu.sync_copy(
                acc_sh.at[pl.ds(_N_ABLK * _ZROWS, _A_TAIL)],
                out_hbm.at[pl.ds(lo + _N_ABLK * _ZROWS, _A_TAIL)],
            )

    return scatter_kernel(vg, dst2d)


def _node_mlp(parts, w3, b3, w4, b4):
    n_tile = 2000
    grid = (_N_NODES // n_tile,)

    def body(p0_ref, p1_ref, w3_ref, b3_ref, w4_ref, b4_ref, out_ref):
        v = p0_ref[...] + p1_ref[...]
        h = jnp.maximum(
            jnp.dot(v, w3_ref[...], preferred_element_type=jnp.float32) + b3_ref[...],
            0.0,
        )
        out_ref[...] = (
            jnp.dot(h, w4_ref[...], preferred_element_type=jnp.float32) + b4_ref[...]
        )

    return pl.pallas_call(
        body,
        grid=grid,
        in_specs=[
            pl.BlockSpec((n_tile, _D), lambda i: (i, 0)),
            pl.BlockSpec((n_tile, _D), lambda i: (i, 0)),
            pl.BlockSpec((_D, _D), lambda i: (0, 0)),
            pl.BlockSpec((1, _D), lambda i: (0, 0)),
            pl.BlockSpec((_D, _D), lambda i: (0, 0)),
            pl.BlockSpec((1, _D), lambda i: (0, 0)),
        ],
        out_specs=pl.BlockSpec((n_tile, _D), lambda i: (i, 0)),
        out_shape=jax.ShapeDtypeStruct((_N_NODES, _D), jnp.float32),
    )(parts[0], parts[1], w3, b3, w4, b4)


def kernel(x, edge_index, edge_attr, W1, b1, W2, b2, W3, b3, W4, b4):
    x2d = x.reshape(_N_EDGES, _D)
    w1a = W1[:4]
    w1b = W1[4:]
    vg = [
        _edge_mlp(
            x2d,
            edge_attr.T,
            w1a,
            w1b,
            b1.reshape(1, _D),
            W2,
            b2.reshape(1, _D),
            h,
        )
        for h in range(_N_SLICES)
    ]
    dst4 = edge_index[1].reshape(_N_SLICES, _N_CHUNKS, _IDX_ROWS, 128)
    parts = [_scatter_sc(vg[h], dst4[h]) for h in range(_N_SLICES)]
    out = _node_mlp(parts, W3, b3.reshape(1, _D), W4, b4.reshape(1, _D))
    return out.reshape(1, _N_NODES, _D)


# speedup vs baseline: 1.6839x; 1.0160x over previous
import functools

import jax
import jax.numpy as jnp
from jax import lax
from jax.experimental import pallas as pl
from jax.experimental.pallas import tpu as pltpu
from jax.experimental.pallas import tpu_sc as plsc

_N_NODES = 10000
_N_EDGES = 320000
_D = 128

_NC = 2
_NS = 16
_NW = _NC * _NS

_N_SLICES = 2
_E_HALF = _N_EDGES // _N_SLICES
_CHUNK = 256
_IDX_ROWS = _CHUNK // 128
_N_CHUNKS = _E_HALF // _CHUNK
_NBUF = 2
_HALF = _N_NODES // 2
_TRASH = _HALF
_ACC_ROWS = _HALF + 16
_ZROWS = 128
_N_ABLK = _HALF // _ZROWS
_A_TAIL = _HALF % _ZROWS


def _edge_mlp(x2d, ea_t, w1a, w1b, b1, w2, b2, half):
    e_tile = 16000
    grid = (_E_HALF // e_tile,)
    off = half * (_E_HALF // e_tile)

    def body(x_ref, ea_ref, w1a_ref, w1b_ref, b1_ref, w2_ref, b2_ref, out_ref):
        h = lax.dot_general(
            ea_ref[...],
            w1a_ref[...],
            dimension_numbers=(((0,), (0,)), ((), ())),
            preferred_element_type=jnp.float32,
        )
        h = h + jnp.dot(
            x_ref[...].astype(jnp.bfloat16),
            w1b_ref[...].astype(jnp.bfloat16),
            preferred_element_type=jnp.float32,
        )
        h = jnp.maximum(h + b1_ref[...], 0.0)
        out_ref[...] = (
            jnp.dot(
                h.astype(jnp.bfloat16),
                w2_ref[...].astype(jnp.bfloat16),
                preferred_element_type=jnp.float32,
            )
            + b2_ref[...]
        )

    return pl.pallas_call(
        body,
        grid=grid,
        in_specs=[
            pl.BlockSpec((e_tile, _D), lambda i: (i + off, 0)),
            pl.BlockSpec((4, e_tile), lambda i: (0, i + off)),
            pl.BlockSpec((4, _D), lambda i: (0, 0)),
            pl.BlockSpec((_D, _D), lambda i: (0, 0)),
            pl.BlockSpec((1, _D), lambda i: (0, 0)),
            pl.BlockSpec((_D, _D), lambda i: (0, 0)),
            pl.BlockSpec((1, _D), lambda i: (0, 0)),
        ],
        out_specs=pl.BlockSpec((e_tile, _D), lambda i: (i, 0)),
        out_shape=jax.ShapeDtypeStruct((_E_HALF, _D), jnp.float32),
    )(x2d, ea_t, w1a, w1b, b1, w2, b2)


def _scatter_sc(vg, dst2d):
    mesh = plsc.VectorSubcoreMesh(core_axis_name="c", subcore_axis_name="s")

    @functools.partial(
        pl.kernel,
        out_type=jax.ShapeDtypeStruct((_N_NODES, _D), jnp.float32),
        mesh=mesh,
        scratch_types=[
            pltpu.VMEM((_NBUF, _IDX_ROWS, 128), jnp.int32),
            pltpu.VMEM((_NBUF, _CHUNK, _D), jnp.float32),
            pltpu.VMEM_SHARED((_ACC_ROWS, _D), jnp.float32),
            pltpu.SemaphoreType.DMA,
            pltpu.SemaphoreType.DMA,
            pltpu.SemaphoreType.DMA,
            pltpu.SemaphoreType.DMA,
            pltpu.SemaphoreType.DMA,
            pltpu.SemaphoreType.DMA,
        ],
    )
    def scatter_kernel(
        vg_hbm, dst_hbm, out_hbm, idx_v, rows_v, acc_sh, r0, r1, r2, s0, s1, s2
    ):
        cid = lax.axis_index("c")
        sid = lax.axis_index("s")
        lo = cid * _HALF
        rsem = [r0, r1, r2]
        ssem = [s0, s1, s2]

        zbuf = rows_v.at[0].at[pl.ds(0, _ZROWS)]
        zv = jnp.zeros((16,), jnp.float32)

        def zrow(r, carry):
            for c16 in range(_D // 16):
                rows_v[0, r, pl.ds(c16 * 16, 16)] = zv
            return carry

        lax.fori_loop(0, _ZROWS, zrow, 0)

        n_blk_iters = -(-_N_ABLK // _NS)

        def zblk(it, carry):
            b = it * _NS + sid

            @pl.when(b < _N_ABLK)
            def _():
                pltpu.sync_copy(zbuf, acc_sh.at[pl.ds(b * _ZROWS, _ZROWS)])

            return carry

        lax.fori_loop(0, n_blk_iters, zblk, 0)

        @pl.when(sid == 0)
        def _():
            pltpu.sync_copy(
                zbuf.at[pl.ds(0, _A_TAIL)],
                acc_sh.at[pl.ds(_N_ABLK * _ZROWS, _A_TAIL)],
            )

        n_extra = _N_CHUNKS - (_N_CHUNKS // _NS) * _NS
        n_i = jnp.where(sid < n_extra, _N_CHUNKS // _NS + 1, _N_CHUNKS // _NS)

        def issue_reads(k, b):
            chunk = k * _NS + sid
            pltpu.async_copy(dst_hbm.at[chunk], idx_v.at[b], rsem[b])
            pltpu.async_copy(
                vg_hbm.at[pl.ds(chunk * _CHUNK, _CHUNK)], rows_v.at[b], rsem[b]
            )

        def wait_reads(k, b):
            chunk = k * _NS + sid
            pltpu.make_async_copy(dst_hbm.at[chunk], idx_v.at[b], rsem[b]).wait()
            pltpu.make_async_copy(
                vg_hbm.at[pl.ds(chunk * _CHUNK, _CHUNK)], rows_v.at[b], rsem[b]
            ).wait()

        def drain_adds(b):
            for j in range(_IDX_ROWS):
                pltpu.make_async_copy(
                    rows_v.at[b].at[pl.ds(j * 128, 128)],
                    acc_sh.at[idx_v.at[b].at[j]],
                    ssem[b],
                ).wait()

        issue_reads(0, 0)
        issue_reads(1, 1)
        plsc.subcore_barrier()

        trash = _TRASH + lax.iota(jnp.int32, 16)
        n_slots = -(-(_N_CHUNKS // _NS + 1) // _NBUF)

        def slot_body(it3, carry):
            for b in range(_NBUF):
                k = it3 * _NBUF + b
                b2 = (b + 2) % _NBUF

                @pl.when(k < n_i)
                def _():
                    wait_reads(k, b)
                    for j in range(_IDX_ROWS):
                        for c16 in range(128 // 16):
                            v = idx_v[b, j, pl.ds(c16 * 16, 16)]
                            local = v - lo
                            ok = (local >= 0) & (local < _HALF)
                            idx_v[b, j, pl.ds(c16 * 16, 16)] = jnp.where(
                                ok, local, trash
                            )
                    for j in range(_IDX_ROWS):
                        pltpu.async_copy(
                            rows_v.at[b].at[pl.ds(j * 128, 128)],
                            acc_sh.at[idx_v.at[b].at[j]],
                            ssem[b],
                            add=True,
                        )

                drain_guard = (
                    (k + 2 < n_i) if _NBUF == 2 else (k >= 1) & (k + 2 < n_i)
                )

                @pl.when(drain_guard)
                def _():
                    drain_adds(b2)

                @pl.when(k + 2 < n_i)
                def _():
                    issue_reads(k + 2, b2)

            return carry

        lax.fori_loop(0, n_slots, slot_body, 0)
        for b in range(_NBUF):
            drain_adds(b)
        plsc.subcore_barrier()

        def wblk(it, carry):
            b = it * _NS + sid

            @pl.when(b < _N_ABLK)
            def _():
                pltpu.sync_copy(
                    acc_sh.at[pl.ds(b * _ZROWS, _ZROWS)],
                    out_hbm.at[pl.ds(lo + b * _ZROWS, _ZROWS)],
                )

            return carry

        lax.fori_loop(0, n_blk_iters, wblk, 0)

        @pl.when(sid == 0)
        def _():
            pltpu.sync_copy(
                acc_sh.at[pl.ds(_N_ABLK * _ZROWS, _A_TAIL)],
                out_hbm.at[pl.ds(lo + _N_ABLK * _ZROWS, _A_TAIL)],
            )

    return scatter_kernel(vg, dst2d)


def _node_mlp(parts, w3, b3, w4, b4):
    n_tile = 2000
    grid = (_N_NODES // n_tile,)

    def body(p0_ref, p1_ref, w3_ref, b3_ref, w4_ref, b4_ref, out_ref):
        v = p0_ref[...] + p1_ref[...]
        h = jnp.maximum(
            jnp.dot(v, w3_ref[...], preferred_element_type=jnp.float32) + b3_ref[...],
            0.0,
        )
        out_ref[...] = (
            jnp.dot(h, w4_ref[...], preferred_element_type=jnp.float32) + b4_ref[...]
        )

    return pl.pallas_call(
        body,
        grid=grid,
        in_specs=[
            pl.BlockSpec((n_tile, _D), lambda i: (i, 0)),
            pl.BlockSpec((n_tile, _D), lambda i: (i, 0)),
            pl.BlockSpec((_D, _D), lambda i: (0, 0)),
            pl.BlockSpec((1, _D), lambda i: (0, 0)),
            pl.BlockSpec((_D, _D), lambda i: (0, 0)),
            pl.BlockSpec((1, _D), lambda i: (0, 0)),
        ],
        out_specs=pl.BlockSpec((n_tile, _D), lambda i: (i, 0)),
        out_shape=jax.ShapeDtypeStruct((_N_NODES, _D), jnp.float32),
    )(parts[0], parts[1], w3, b3, w4, b4)


def kernel(x, edge_index, edge_attr, W1, b1, W2, b2, W3, b3, W4, b4):
    x2d = x.reshape(_N_EDGES, _D)
    w1a = W1[:4]
    w1b = W1[4:]
    vg = [
        _edge_mlp(
            x2d,
            edge_attr.T,
            w1a,
            w1b,
            b1.reshape(1, _D),
            W2,
            b2.reshape(1, _D),
            h,
        )
        for h in range(_N_SLICES)
    ]
    dst4 = edge_index[1].reshape(_N_SLICES, _N_CHUNKS, _IDX_ROWS, 128)
    parts = [_scatter_sc(vg[h], dst4[h]) for h in range(_N_SLICES)]
    out = _node_mlp(parts, W3, b3.reshape(1, _D), W4, b4.reshape(1, _D))
    return out.reshape(1, _N_NODES, _D)
